# baseline (speedup 1.0000x reference)
import functools

import jax
import jax.numpy as jnp
from jax import lax
from jax.experimental import pallas as pl
from jax.experimental.pallas import tpu as pltpu
from jax.experimental.pallas import tpu_sc as plsc

_NC, _NS = 2, 16
_NW = _NC * _NS


def kernel(rule_prob, token_prob, reference_prob, length):
    L, B, R = rule_prob.shape
    nb = B // _NW
    mesh = plsc.VectorSubcoreMesh(core_axis_name="c", subcore_axis_name="s")

    @functools.partial(
        pl.kernel,
        out_type=jax.ShapeDtypeStruct((B, R), jnp.float32),
        mesh=mesh,
        scratch_types=[
            pltpu.VMEM((16,), jnp.int32),
            pltpu.VMEM((nb, R), jnp.float32),
            pltpu.SemaphoreType.DMA,
        ],
    )
    def sc_copy(len_hbm, rule_hbm, out_hbm, len_v, buf, sem):
        w = lax.axis_index("s") * _NC + lax.axis_index("c")
        pltpu.sync_copy(len_hbm, len_v.at[pl.ds(0, 1)])
        idx = len_v[...][0] - 1
        base = w * nb
        pltpu.async_copy(rule_hbm.at[idx, pl.ds(base, nb)], buf, sem).wait()
        pltpu.sync_copy(buf, out_hbm.at[pl.ds(base, nb)])

    r = sc_copy(length, rule_prob)
    idx = length[0] - 1
    t = jnp.take(token_prob, idx, axis=0)
    p = jnp.take(reference_prob, idx, axis=0)
    return (r, t, p)
